# Initial kernel scaffold; baseline (speedup 1.0000x reference)
import jax
import jax.numpy as jnp
from jax.experimental import pallas as pl


def kernel(x, hyperedge_index, W, b):
    raise NotImplementedError("write your pallas kernel here")


# speedup vs baseline: 5.1097x; 5.1097x over previous
import functools

import jax
import jax.numpy as jnp
from jax import lax
from jax.experimental import pallas as pl
from jax.experimental.pallas import tpu as pltpu
from jax.experimental.pallas import tpu_sc as plsc

NV = 10000
NNZ = 160000
D = 256
DH = 128
NC = 2
NS = 16
NVP = 10240
DR = NVP // 128
CHUNK = 80
IBLK = 5
IROWS = 25
CPT = NNZ // CHUNK // NS
RPT = NVP // NS
ZROWS = 64


@functools.lru_cache(maxsize=None)
def _make_sc_stage(with_degs: bool):
    mesh = plsc.VectorSubcoreMesh(
        core_axis_name="c", subcore_axis_name="s",
        num_cores=NC, num_subcores=NS)
    out_type = [jax.ShapeDtypeStruct((NC, NVP, DH), jnp.float32)]
    scratch = [
        pltpu.VMEM((IROWS, CHUNK), jnp.int32),
        pltpu.VMEM((IROWS, CHUNK), jnp.int32),
        pltpu.VMEM((CHUNK, DH), jnp.float32),
        pltpu.VMEM((ZROWS, DH), jnp.float32),
        pltpu.VMEM_SHARED((NVP, DH), jnp.float32),
    ]
    if with_degs:
        out_type += [jax.ShapeDtypeStruct((NVP,), jnp.float32),
                     jax.ShapeDtypeStruct((NVP,), jnp.float32)]
        scratch += [
            pltpu.VMEM((CHUNK,), jnp.float32),
            pltpu.VMEM((RPT,), jnp.float32),
            pltpu.VMEM_SHARED((NVP,), jnp.float32),
        ]

    @functools.partial(pl.kernel, mesh=mesh, out_type=out_type,
                       scratch_types=scratch)
    def stage(*refs):
        if with_degs:
            (table, gidx, sidx, z128, ones1d, z1d,
             out, edeg_out, vdeg_out,
             gbuf, sbuf, rows, zbuf, acc, ones_v, dsum, dacc) = refs
        else:
            (table, gidx, sidx, z128,
             out, gbuf, sbuf, rows, zbuf, acc) = refs
        c = lax.axis_index("c")
        s = lax.axis_index("s")
        pltpu.sync_copy(z128, zbuf)
        base = s * RPT
        for i in range(RPT // ZROWS):
            pltpu.sync_copy(zbuf, acc.at[pl.ds(base + i * ZROWS, ZROWS)])
        if with_degs:
            pltpu.sync_copy(ones1d, ones_v)

            @pl.when(s == 0)
            def _():
                pltpu.sync_copy(z1d, dacc)
        plsc.subcore_barrier()

        def blk_body(blk, carry):
            pltpu.sync_copy(gidx.at[s].at[blk], gbuf)
            pltpu.sync_copy(sidx.at[s].at[blk], sbuf)

            def chunk_body(j, carry2):
                pltpu.sync_copy(table.at[c].at[gbuf.at[j]], rows)
                pltpu.sync_copy(rows, acc.at[sbuf.at[j]], add=True)
                if with_degs:
                    @pl.when(c == 0)
                    def _():
                        pltpu.sync_copy(ones_v, dacc.at[sbuf.at[j]],
                                        add=True)

                    @pl.when(c == 1)
                    def _():
                        pltpu.sync_copy(ones_v, dacc.at[gbuf.at[j]],
                                        add=True)
                return carry2

            lax.fori_loop(0, IROWS, chunk_body, 0)
            return carry

        lax.fori_loop(0, IBLK, blk_body, 0)
        plsc.subcore_barrier()

        for i in range(RPT // ZROWS):
            pltpu.sync_copy(acc.at[pl.ds(base + i * ZROWS, ZROWS)], zbuf)
            pltpu.sync_copy(zbuf,
                            out.at[c].at[pl.ds(base + i * ZROWS, ZROWS)])
        if with_degs:
            pltpu.sync_copy(dacc.at[pl.ds(base, RPT)], dsum)

            @pl.when(c == 0)
            def _():
                pltpu.sync_copy(dsum, edeg_out.at[pl.ds(base, RPT)])

            @pl.when(c == 1)
            def _():
                pltpu.sync_copy(dsum, vdeg_out.at[pl.ds(base, RPT)])

    return stage


_MM_B = 400


def _matmul_body(x_ref, w_ref, b_ref, out_ref):
    r = jnp.dot(x_ref[...], w_ref[...],
                preferred_element_type=jnp.float32) + b_ref[...]
    out_ref[0, :, :] = r[:, :DH]
    out_ref[1, :, :] = r[:, DH:]


def _matmul_planes(x, W, b2):
    return pl.pallas_call(
        _matmul_body,
        grid=(NV // _MM_B,),
        in_specs=[
            pl.BlockSpec((_MM_B, D), lambda i: (i, 0)),
            pl.BlockSpec((D, D), lambda i: (0, 0)),
            pl.BlockSpec((1, D), lambda i: (0, 0)),
        ],
        out_specs=pl.BlockSpec((NC, _MM_B, DH), lambda i: (0, i, 0)),
        out_shape=jax.ShapeDtypeStruct((NC, NVP, DH), jnp.float32),
    )(x, W, b2)


_DIV_B = 1024
_OUT_B = 1000


def _div_mid_body(sum_ref, deg_ref, out_ref):
    r = 1.0 / jnp.maximum(deg_ref[...], 1.0)
    out_ref[0, :, :] = sum_ref[0, :, :] * r
    out_ref[1, :, :] = sum_ref[1, :, :] * r


def _div_mid(sums, deg_col):
    return pl.pallas_call(
        _div_mid_body,
        grid=(NVP // _DIV_B,),
        in_specs=[
            pl.BlockSpec((NC, _DIV_B, DH), lambda i: (0, i, 0)),
            pl.BlockSpec((_DIV_B, 1), lambda i: (i, 0)),
        ],
        out_specs=pl.BlockSpec((NC, _DIV_B, DH), lambda i: (0, i, 0)),
        out_shape=jax.ShapeDtypeStruct((NC, NVP, DH), jnp.float32),
    )(sums, deg_col)


def _div_final_body(sum_ref, deg_ref, out_ref):
    r = 1.0 / jnp.maximum(deg_ref[...], 1.0)
    out_ref[:, :DH] = sum_ref[0, :, :] * r
    out_ref[:, DH:] = sum_ref[1, :, :] * r


def _div_final(sums, deg_col):
    return pl.pallas_call(
        _div_final_body,
        grid=(NV // _OUT_B,),
        in_specs=[
            pl.BlockSpec((NC, _OUT_B, DH), lambda i: (0, i, 0)),
            pl.BlockSpec((_OUT_B, 1), lambda i: (i, 0)),
        ],
        out_specs=pl.BlockSpec((_OUT_B, D), lambda i: (i, 0)),
        out_shape=jax.ShapeDtypeStruct((NV, D), jnp.float32),
    )(sums, deg_col)


def kernel(x, hyperedge_index, W, b):
    v4d = hyperedge_index[0].reshape(NS, IBLK, IROWS, CHUNK)
    e4d = hyperedge_index[1].reshape(NS, IBLK, IROWS, CHUNK)
    z128 = jnp.zeros((ZROWS, DH), jnp.float32)
    ones1d = jnp.ones((CHUNK,), jnp.float32)
    z1d = jnp.zeros((NVP,), jnp.float32)

    xt = _matmul_planes(x, W, b.reshape(1, D))
    e_sum, e_deg, v_deg = _make_sc_stage(True)(xt, v4d, e4d, z128, ones1d, z1d)
    e_feat = _div_mid(e_sum, e_deg.reshape(NVP, 1))
    (v_sum,) = _make_sc_stage(False)(e_feat, e4d, v4d, z128)
    return _div_final(v_sum, v_deg.reshape(NVP, 1))


# speedup vs baseline: 6.3355x; 1.2399x over previous
import functools

import jax
import jax.numpy as jnp
from jax import lax
from jax.experimental import pallas as pl
from jax.experimental.pallas import tpu as pltpu
from jax.experimental.pallas import tpu_sc as plsc

NV = 10000
NNZ = 160000
D = 256
DH = 128
NC = 2
NS = 16
NVP = 10240
DR = NVP // 128
CHUNK = 80
IBLK = 5
IROWS = 25
CPT = NNZ // CHUNK // NS
RPT = NVP // NS
ZROWS = 64


@functools.lru_cache(maxsize=None)
def _make_sc_stage(with_degs: bool):
    mesh = plsc.VectorSubcoreMesh(
        core_axis_name="c", subcore_axis_name="s",
        num_cores=NC, num_subcores=NS)
    out_type = [jax.ShapeDtypeStruct((NC, NVP, DH), jnp.float32)]
    scratch = [
        pltpu.VMEM((IROWS, CHUNK), jnp.int32),
        pltpu.VMEM((IROWS, CHUNK), jnp.int32),
        pltpu.VMEM((CHUNK, DH), jnp.float32),
        pltpu.VMEM((CHUNK, DH), jnp.float32),
        pltpu.VMEM((ZROWS, DH), jnp.float32),
        pltpu.VMEM_SHARED((NVP, DH), jnp.float32),
        pltpu.SemaphoreType.DMA,
        pltpu.SemaphoreType.DMA,
        pltpu.SemaphoreType.DMA,
        pltpu.SemaphoreType.DMA,
    ]
    if with_degs:
        out_type += [jax.ShapeDtypeStruct((NVP,), jnp.float32),
                     jax.ShapeDtypeStruct((NVP,), jnp.float32)]
        scratch += [
            pltpu.VMEM((CHUNK,), jnp.float32),
            pltpu.VMEM((RPT,), jnp.float32),
            pltpu.VMEM_SHARED((NVP,), jnp.float32),
            pltpu.SemaphoreType.DMA,
        ]

    @functools.partial(pl.kernel, mesh=mesh, out_type=out_type,
                       scratch_types=scratch)
    def stage(*refs):
        if with_degs:
            (table, gidx, sidx, z128, ones1d, z1d,
             out, edeg_out, vdeg_out,
             gbuf, sbuf, rows_a, rows_b, zbuf, acc, gs_a, gs_b, ss_a, ss_b,
             ones_v, dsum, dacc, dsem) = refs
        else:
            (table, gidx, sidx, z128,
             out, gbuf, sbuf, rows_a, rows_b, zbuf, acc,
             gs_a, gs_b, ss_a, ss_b) = refs
        c = lax.axis_index("c")
        s = lax.axis_index("s")
        pltpu.sync_copy(z128, zbuf)
        base = s * RPT
        for i in range(RPT // ZROWS):
            pltpu.sync_copy(zbuf, acc.at[pl.ds(base + i * ZROWS, ZROWS)])
        if with_degs:
            pltpu.sync_copy(ones1d, ones_v)

            @pl.when(s == 0)
            def _():
                pltpu.sync_copy(z1d, dacc)
        plsc.subcore_barrier()

        def deg_scatter(j):
            if not with_degs:
                return

            @pl.when(c == 0)
            def _():
                pltpu.async_copy(ones_v, dacc.at[sbuf.at[j]], dsem,
                                 add=True).wait()

            @pl.when(c == 1)
            def _():
                pltpu.async_copy(ones_v, dacc.at[gbuf.at[j]], dsem,
                                 add=True).wait()

        def blk_body(blk, carry):
            pltpu.sync_copy(gidx.at[s].at[blk], gbuf)
            pltpu.sync_copy(sidx.at[s].at[blk], sbuf)

            pltpu.async_copy(table.at[c].at[gbuf.at[0]], rows_a, gs_a).wait()
            pltpu.async_copy(rows_a, acc.at[sbuf.at[0]], ss_a,
                             add=True).wait()
            deg_scatter(0)

            def duo_body(t, carry2):
                ja = 2 * t + 1
                jb = 2 * t + 2
                ga = pltpu.async_copy(table.at[c].at[gbuf.at[ja]],
                                      rows_a, gs_a)
                gb = pltpu.async_copy(table.at[c].at[gbuf.at[jb]],
                                      rows_b, gs_b)
                ga.wait()
                sa = pltpu.async_copy(rows_a, acc.at[sbuf.at[ja]], ss_a,
                                      add=True)
                gb.wait()
                sb = pltpu.async_copy(rows_b, acc.at[sbuf.at[jb]], ss_b,
                                      add=True)
                deg_scatter(ja)
                deg_scatter(jb)
                sa.wait()
                sb.wait()
                return carry2

            lax.fori_loop(0, (IROWS - 1) // 2, duo_body, 0)
            return carry

        lax.fori_loop(0, IBLK, blk_body, 0)
        plsc.subcore_barrier()

        for i in range(RPT // ZROWS):
            pltpu.sync_copy(acc.at[pl.ds(base + i * ZROWS, ZROWS)], zbuf)
            pltpu.sync_copy(zbuf,
                            out.at[c].at[pl.ds(base + i * ZROWS, ZROWS)])
        if with_degs:
            pltpu.sync_copy(dacc.at[pl.ds(base, RPT)], dsum)

            @pl.when(c == 0)
            def _():
                pltpu.sync_copy(dsum, edeg_out.at[pl.ds(base, RPT)])

            @pl.when(c == 1)
            def _():
                pltpu.sync_copy(dsum, vdeg_out.at[pl.ds(base, RPT)])

    return stage


_MM_B = 400


def _matmul_body(x_ref, w_ref, b_ref, out_ref):
    r = jnp.dot(x_ref[...], w_ref[...],
                preferred_element_type=jnp.float32) + b_ref[...]
    out_ref[0, :, :] = r[:, :DH]
    out_ref[1, :, :] = r[:, DH:]


def _matmul_planes(x, W, b2):
    return pl.pallas_call(
        _matmul_body,
        grid=(NV // _MM_B,),
        in_specs=[
            pl.BlockSpec((_MM_B, D), lambda i: (i, 0)),
            pl.BlockSpec((D, D), lambda i: (0, 0)),
            pl.BlockSpec((1, D), lambda i: (0, 0)),
        ],
        out_specs=pl.BlockSpec((NC, _MM_B, DH), lambda i: (0, i, 0)),
        out_shape=jax.ShapeDtypeStruct((NC, NVP, DH), jnp.float32),
    )(x, W, b2)


_DIV_B = 1024
_OUT_B = 1000


def _div_mid_body(sum_ref, deg_ref, out_ref):
    r = 1.0 / jnp.maximum(deg_ref[...], 1.0)
    out_ref[0, :, :] = sum_ref[0, :, :] * r
    out_ref[1, :, :] = sum_ref[1, :, :] * r


def _div_mid(sums, deg_col):
    return pl.pallas_call(
        _div_mid_body,
        grid=(NVP // _DIV_B,),
        in_specs=[
            pl.BlockSpec((NC, _DIV_B, DH), lambda i: (0, i, 0)),
            pl.BlockSpec((_DIV_B, 1), lambda i: (i, 0)),
        ],
        out_specs=pl.BlockSpec((NC, _DIV_B, DH), lambda i: (0, i, 0)),
        out_shape=jax.ShapeDtypeStruct((NC, NVP, DH), jnp.float32),
    )(sums, deg_col)


def _div_final_body(sum_ref, deg_ref, out_ref):
    r = 1.0 / jnp.maximum(deg_ref[...], 1.0)
    out_ref[:, :DH] = sum_ref[0, :, :] * r
    out_ref[:, DH:] = sum_ref[1, :, :] * r


def _div_final(sums, deg_col):
    return pl.pallas_call(
        _div_final_body,
        grid=(NV // _OUT_B,),
        in_specs=[
            pl.BlockSpec((NC, _OUT_B, DH), lambda i: (0, i, 0)),
            pl.BlockSpec((_OUT_B, 1), lambda i: (i, 0)),
        ],
        out_specs=pl.BlockSpec((_OUT_B, D), lambda i: (i, 0)),
        out_shape=jax.ShapeDtypeStruct((NV, D), jnp.float32),
    )(sums, deg_col)


def kernel(x, hyperedge_index, W, b):
    v4d = hyperedge_index[0].reshape(NS, IBLK, IROWS, CHUNK)
    e4d = hyperedge_index[1].reshape(NS, IBLK, IROWS, CHUNK)
    z128 = jnp.zeros((ZROWS, DH), jnp.float32)
    ones1d = jnp.ones((CHUNK,), jnp.float32)
    z1d = jnp.zeros((NVP,), jnp.float32)

    xt = _matmul_planes(x, W, b.reshape(1, D))
    e_sum, e_deg, v_deg = _make_sc_stage(True)(xt, v4d, e4d, z128, ones1d, z1d)
    e_feat = _div_mid(e_sum, e_deg.reshape(NVP, 1))
    (v_sum,) = _make_sc_stage(False)(e_feat, e4d, v4d, z128)
    return _div_final(v_sum, v_deg.reshape(NVP, 1))


# speedup vs baseline: 6.5582x; 1.0351x over previous
import functools

import jax
import jax.numpy as jnp
from jax import lax
from jax.experimental import pallas as pl
from jax.experimental.pallas import tpu as pltpu
from jax.experimental.pallas import tpu_sc as plsc

NV = 10000
NNZ = 160000
D = 256
DH = 128
NC = 2
NS = 16
NVP = 10240
DR = NVP // 128
CHUNK = 80
IBLK = 5
IROWS = 25
CPT = NNZ // CHUNK // NS
RPT = NVP // NS
ZROWS = 32


@functools.lru_cache(maxsize=None)
def _make_sc_stage(with_degs: bool):
    mesh = plsc.VectorSubcoreMesh(
        core_axis_name="c", subcore_axis_name="s",
        num_cores=NC, num_subcores=NS)
    out_type = [jax.ShapeDtypeStruct((NC, NVP, DH), jnp.float32)]
    scratch = [
        pltpu.VMEM((IROWS, CHUNK), jnp.int32),
        pltpu.VMEM((IROWS, CHUNK), jnp.int32),
        pltpu.VMEM((CHUNK, DH), jnp.float32),
        pltpu.VMEM((CHUNK, DH), jnp.float32),
        pltpu.VMEM((CHUNK, DH), jnp.float32),
        pltpu.VMEM((ZROWS, DH), jnp.float32),
        pltpu.VMEM_SHARED((NVP, DH), jnp.float32),
        pltpu.SemaphoreType.DMA,
        pltpu.SemaphoreType.DMA,
        pltpu.SemaphoreType.DMA,
        pltpu.SemaphoreType.DMA,
        pltpu.SemaphoreType.DMA,
        pltpu.SemaphoreType.DMA,
    ]
    if with_degs:
        out_type += [jax.ShapeDtypeStruct((NVP,), jnp.float32),
                     jax.ShapeDtypeStruct((NVP,), jnp.float32)]
        scratch += [
            pltpu.VMEM((CHUNK,), jnp.float32),
            pltpu.VMEM((RPT,), jnp.float32),
            pltpu.VMEM_SHARED((NVP,), jnp.float32),
            pltpu.SemaphoreType.DMA,
        ]

    @functools.partial(pl.kernel, mesh=mesh, out_type=out_type,
                       scratch_types=scratch)
    def stage(*refs):
        if with_degs:
            (table, gidx, sidx, z128, ones1d, z1d,
             out, edeg_out, vdeg_out,
             gbuf, sbuf, rows_a, rows_b, rows_c, zbuf, acc,
             gs_a, gs_b, gs_c, ss_a, ss_b, ss_c,
             ones_v, dsum, dacc, dsem) = refs
        else:
            (table, gidx, sidx, z128,
             out, gbuf, sbuf, rows_a, rows_b, rows_c, zbuf, acc,
             gs_a, gs_b, gs_c, ss_a, ss_b, ss_c) = refs
        c = lax.axis_index("c")
        s = lax.axis_index("s")
        pltpu.sync_copy(z128, zbuf)
        base = s * RPT
        for i in range(RPT // ZROWS):
            pltpu.sync_copy(zbuf, acc.at[pl.ds(base + i * ZROWS, ZROWS)])
        if with_degs:
            pltpu.sync_copy(ones1d, ones_v)

            @pl.when(s == 0)
            def _():
                pltpu.sync_copy(z1d, dacc)
        plsc.subcore_barrier()

        def deg_scatter(j):
            if not with_degs:
                return

            @pl.when(c == 0)
            def _():
                pltpu.async_copy(ones_v, dacc.at[sbuf.at[j]], dsem,
                                 add=True).wait()

            @pl.when(c == 1)
            def _():
                pltpu.async_copy(ones_v, dacc.at[gbuf.at[j]], dsem,
                                 add=True).wait()

        def blk_body(blk, carry):
            pltpu.sync_copy(gidx.at[s].at[blk], gbuf)
            pltpu.sync_copy(sidx.at[s].at[blk], sbuf)

            pltpu.async_copy(table.at[c].at[gbuf.at[0]], rows_a, gs_a).wait()
            pltpu.async_copy(rows_a, acc.at[sbuf.at[0]], ss_a,
                             add=True).wait()
            deg_scatter(0)

            def trio_body(t, carry2):
                ja = 3 * t + 1
                jb = 3 * t + 2
                jc = 3 * t + 3
                ga = pltpu.async_copy(table.at[c].at[gbuf.at[ja]],
                                      rows_a, gs_a)
                gb = pltpu.async_copy(table.at[c].at[gbuf.at[jb]],
                                      rows_b, gs_b)
                gc = pltpu.async_copy(table.at[c].at[gbuf.at[jc]],
                                      rows_c, gs_c)
                ga.wait()
                sa = pltpu.async_copy(rows_a, acc.at[sbuf.at[ja]], ss_a,
                                      add=True)
                gb.wait()
                sb = pltpu.async_copy(rows_b, acc.at[sbuf.at[jb]], ss_b,
                                      add=True)
                gc.wait()
                sc = pltpu.async_copy(rows_c, acc.at[sbuf.at[jc]], ss_c,
                                      add=True)
                deg_scatter(ja)
                deg_scatter(jb)
                deg_scatter(jc)
                sa.wait()
                sb.wait()
                sc.wait()
                return carry2

            lax.fori_loop(0, (IROWS - 1) // 3, trio_body, 0)
            return carry

        lax.fori_loop(0, IBLK, blk_body, 0)
        plsc.subcore_barrier()

        for i in range(RPT // ZROWS):
            pltpu.sync_copy(acc.at[pl.ds(base + i * ZROWS, ZROWS)], zbuf)
            pltpu.sync_copy(zbuf,
                            out.at[c].at[pl.ds(base + i * ZROWS, ZROWS)])
        if with_degs:
            pltpu.sync_copy(dacc.at[pl.ds(base, RPT)], dsum)

            @pl.when(c == 0)
            def _():
                pltpu.sync_copy(dsum, edeg_out.at[pl.ds(base, RPT)])

            @pl.when(c == 1)
            def _():
                pltpu.sync_copy(dsum, vdeg_out.at[pl.ds(base, RPT)])

    return stage


_MM_B = 400


def _matmul_body(x_ref, w_ref, b_ref, out_ref):
    r = jnp.dot(x_ref[...], w_ref[...],
                preferred_element_type=jnp.float32) + b_ref[...]
    out_ref[0, :, :] = r[:, :DH]
    out_ref[1, :, :] = r[:, DH:]


def _matmul_planes(x, W, b2):
    return pl.pallas_call(
        _matmul_body,
        grid=(NV // _MM_B,),
        in_specs=[
            pl.BlockSpec((_MM_B, D), lambda i: (i, 0)),
            pl.BlockSpec((D, D), lambda i: (0, 0)),
            pl.BlockSpec((1, D), lambda i: (0, 0)),
        ],
        out_specs=pl.BlockSpec((NC, _MM_B, DH), lambda i: (0, i, 0)),
        out_shape=jax.ShapeDtypeStruct((NC, NVP, DH), jnp.float32),
    )(x, W, b2)


_DIV_B = 1024
_OUT_B = 1000


def _div_mid_body(sum_ref, deg_ref, out_ref):
    r = 1.0 / jnp.maximum(deg_ref[...], 1.0)
    out_ref[0, :, :] = sum_ref[0, :, :] * r
    out_ref[1, :, :] = sum_ref[1, :, :] * r


def _div_mid(sums, deg_col):
    return pl.pallas_call(
        _div_mid_body,
        grid=(NVP // _DIV_B,),
        in_specs=[
            pl.BlockSpec((NC, _DIV_B, DH), lambda i: (0, i, 0)),
            pl.BlockSpec((_DIV_B, 1), lambda i: (i, 0)),
        ],
        out_specs=pl.BlockSpec((NC, _DIV_B, DH), lambda i: (0, i, 0)),
        out_shape=jax.ShapeDtypeStruct((NC, NVP, DH), jnp.float32),
    )(sums, deg_col)


def _div_final_body(sum_ref, deg_ref, out_ref):
    r = 1.0 / jnp.maximum(deg_ref[...], 1.0)
    out_ref[:, :DH] = sum_ref[0, :, :] * r
    out_ref[:, DH:] = sum_ref[1, :, :] * r


def _div_final(sums, deg_col):
    return pl.pallas_call(
        _div_final_body,
        grid=(NV // _OUT_B,),
        in_specs=[
            pl.BlockSpec((NC, _OUT_B, DH), lambda i: (0, i, 0)),
            pl.BlockSpec((_OUT_B, 1), lambda i: (i, 0)),
        ],
        out_specs=pl.BlockSpec((_OUT_B, D), lambda i: (i, 0)),
        out_shape=jax.ShapeDtypeStruct((NV, D), jnp.float32),
    )(sums, deg_col)


def kernel(x, hyperedge_index, W, b):
    v4d = hyperedge_index[0].reshape(NS, IBLK, IROWS, CHUNK)
    e4d = hyperedge_index[1].reshape(NS, IBLK, IROWS, CHUNK)
    z128 = jnp.zeros((ZROWS, DH), jnp.float32)
    ones1d = jnp.ones((CHUNK,), jnp.float32)
    z1d = jnp.zeros((NVP,), jnp.float32)

    xt = _matmul_planes(x, W, b.reshape(1, D))
    e_sum, e_deg, v_deg = _make_sc_stage(True)(xt, v4d, e4d, z128, ones1d, z1d)
    e_feat = _div_mid(e_sum, e_deg.reshape(NVP, 1))
    (v_sum,) = _make_sc_stage(False)(e_feat, e4d, v4d, z128)
    return _div_final(v_sum, v_deg.reshape(NVP, 1))


# speedup vs baseline: 7.4377x; 1.1341x over previous
import functools

import jax
import jax.numpy as jnp
from jax import lax
from jax.experimental import pallas as pl
from jax.experimental.pallas import tpu as pltpu
from jax.experimental.pallas import tpu_sc as plsc

NV = 10000
NNZ = 160000
D = 256
DH = 128
NC = 2
NS = 16
NVP = 10240
DR = NVP // 128
CHUNK = 80
IBLK = 5
IROWS = 25
CPT = NNZ // CHUNK // NS
RPT = NVP // NS
ZROWS = 32


@functools.lru_cache(maxsize=None)
def _make_sc_stage(with_degs: bool):
    mesh = plsc.VectorSubcoreMesh(
        core_axis_name="c", subcore_axis_name="s",
        num_cores=NC, num_subcores=NS)
    out_type = [jax.ShapeDtypeStruct((NC, NVP, DH), jnp.float32)]
    scratch = [
        pltpu.VMEM((IROWS, CHUNK), jnp.int32),
        pltpu.VMEM((IROWS, CHUNK), jnp.int32),
        pltpu.VMEM((CHUNK, DH), jnp.float32),
        pltpu.VMEM((CHUNK, DH), jnp.float32),
        pltpu.VMEM((CHUNK, DH), jnp.float32),
        pltpu.VMEM((ZROWS, DH), jnp.float32),
        pltpu.VMEM_SHARED((NVP, DH), jnp.float32),
        pltpu.SemaphoreType.DMA,
        pltpu.SemaphoreType.DMA,
        pltpu.SemaphoreType.DMA,
        pltpu.SemaphoreType.DMA,
        pltpu.SemaphoreType.DMA,
        pltpu.SemaphoreType.DMA,
    ]
    if with_degs:
        out_type += [jax.ShapeDtypeStruct((NVP,), jnp.float32),
                     jax.ShapeDtypeStruct((NVP,), jnp.float32)]
        scratch += [
            pltpu.VMEM((CHUNK,), jnp.float32),
            pltpu.VMEM((RPT,), jnp.float32),
            pltpu.VMEM_SHARED((NVP,), jnp.float32),
            pltpu.SemaphoreType.DMA,
        ]

    @functools.partial(pl.kernel, mesh=mesh, out_type=out_type,
                       scratch_types=scratch)
    def stage(*refs):
        if with_degs:
            (table, gidx, sidx, z128, ones1d, z1d,
             out, edeg_out, vdeg_out,
             gbuf, sbuf, rows_a, rows_b, rows_c, zbuf, acc,
             gs_a, gs_b, gs_c, ss_a, ss_b, ss_c,
             ones_v, dsum, dacc, dsem) = refs
        else:
            (table, gidx, sidx, z128,
             out, gbuf, sbuf, rows_a, rows_b, rows_c, zbuf, acc,
             gs_a, gs_b, gs_c, ss_a, ss_b, ss_c) = refs
        c = lax.axis_index("c")
        s = lax.axis_index("s")
        pltpu.sync_copy(z128, zbuf)
        base = s * RPT
        for i in range(RPT // ZROWS):
            pltpu.sync_copy(zbuf, acc.at[pl.ds(base + i * ZROWS, ZROWS)])
        if with_degs:
            pltpu.sync_copy(ones1d, ones_v)

            @pl.when(s == 0)
            def _():
                pltpu.sync_copy(z1d, dacc)
        plsc.subcore_barrier()

        def deg_fire(j):
            if not with_degs:
                return

            @pl.when(c == 0)
            def _():
                pltpu.async_copy(ones_v, dacc.at[sbuf.at[j]], dsem, add=True)

            @pl.when(c == 1)
            def _():
                pltpu.async_copy(ones_v, dacc.at[gbuf.at[j]], dsem, add=True)

        def gather(j, buf, sem):
            return pltpu.async_copy(table.at[c].at[gbuf.at[j]], buf, sem)

        def scat(j, buf, sem):
            pltpu.async_copy(buf, acc.at[sbuf.at[j]], sem, add=True)

        def drain_scat(buf, sem):
            pltpu.make_async_copy(buf, acc.at[sbuf.at[0]], sem).wait()

        def blk_body(blk, carry):
            pltpu.sync_copy(gidx.at[s].at[blk], gbuf)
            pltpu.sync_copy(sidx.at[s].at[blk], sbuf)

            g0 = gather(0, rows_a, gs_a)
            g1 = gather(1, rows_b, gs_b)
            g2 = gather(2, rows_c, gs_c)
            g0.wait()
            scat(0, rows_a, ss_a)
            g1.wait()
            scat(1, rows_b, ss_b)
            g2.wait()
            scat(2, rows_c, ss_c)
            deg_fire(0)
            deg_fire(1)
            deg_fire(2)

            def trio_body(t, carry2):
                ja = 3 * t + 3
                jb = 3 * t + 4
                jc = 3 * t + 5
                drain_scat(rows_a, ss_a)
                ga = gather(ja, rows_a, gs_a)
                drain_scat(rows_b, ss_b)
                gb = gather(jb, rows_b, gs_b)
                drain_scat(rows_c, ss_c)
                gc = gather(jc, rows_c, gs_c)
                ga.wait()
                scat(ja, rows_a, ss_a)
                gb.wait()
                scat(jb, rows_b, ss_b)
                gc.wait()
                scat(jc, rows_c, ss_c)
                deg_fire(ja)
                deg_fire(jb)
                deg_fire(jc)
                return carry2

            lax.fori_loop(0, (IROWS - 4) // 3, trio_body, 0)

            drain_scat(rows_a, ss_a)
            gather(IROWS - 1, rows_a, gs_a).wait()
            scat(IROWS - 1, rows_a, ss_a)
            deg_fire(IROWS - 1)
            drain_scat(rows_a, ss_a)
            drain_scat(rows_b, ss_b)
            drain_scat(rows_c, ss_c)
            return carry

        lax.fori_loop(0, IBLK, blk_body, 0)
        if with_degs:
            def deg_drain(j, carry):
                pltpu.make_async_copy(ones_v, dacc.at[sbuf.at[0]],
                                      dsem).wait()
                return carry

            lax.fori_loop(0, CPT, deg_drain, 0)
        plsc.subcore_barrier()

        for i in range(RPT // ZROWS):
            pltpu.sync_copy(acc.at[pl.ds(base + i * ZROWS, ZROWS)], zbuf)
            pltpu.sync_copy(zbuf,
                            out.at[c].at[pl.ds(base + i * ZROWS, ZROWS)])
        if with_degs:
            pltpu.sync_copy(dacc.at[pl.ds(base, RPT)], dsum)

            @pl.when(c == 0)
            def _():
                pltpu.sync_copy(dsum, edeg_out.at[pl.ds(base, RPT)])

            @pl.when(c == 1)
            def _():
                pltpu.sync_copy(dsum, vdeg_out.at[pl.ds(base, RPT)])

    return stage


_MM_B = 400


def _matmul_body(x_ref, w_ref, b_ref, out_ref):
    r = jnp.dot(x_ref[...], w_ref[...],
                preferred_element_type=jnp.float32) + b_ref[...]
    out_ref[0, :, :] = r[:, :DH]
    out_ref[1, :, :] = r[:, DH:]


def _matmul_planes(x, W, b2):
    return pl.pallas_call(
        _matmul_body,
        grid=(NV // _MM_B,),
        in_specs=[
            pl.BlockSpec((_MM_B, D), lambda i: (i, 0)),
            pl.BlockSpec((D, D), lambda i: (0, 0)),
            pl.BlockSpec((1, D), lambda i: (0, 0)),
        ],
        out_specs=pl.BlockSpec((NC, _MM_B, DH), lambda i: (0, i, 0)),
        out_shape=jax.ShapeDtypeStruct((NC, NVP, DH), jnp.float32),
    )(x, W, b2)


_DIV_B = 1024
_OUT_B = 1000


def _div_mid_body(sum_ref, deg_ref, out_ref):
    r = 1.0 / jnp.maximum(deg_ref[...], 1.0)
    out_ref[0, :, :] = sum_ref[0, :, :] * r
    out_ref[1, :, :] = sum_ref[1, :, :] * r


def _div_mid(sums, deg_col):
    return pl.pallas_call(
        _div_mid_body,
        grid=(NVP // _DIV_B,),
        in_specs=[
            pl.BlockSpec((NC, _DIV_B, DH), lambda i: (0, i, 0)),
            pl.BlockSpec((_DIV_B, 1), lambda i: (i, 0)),
        ],
        out_specs=pl.BlockSpec((NC, _DIV_B, DH), lambda i: (0, i, 0)),
        out_shape=jax.ShapeDtypeStruct((NC, NVP, DH), jnp.float32),
    )(sums, deg_col)


def _div_final_body(sum_ref, deg_ref, out_ref):
    r = 1.0 / jnp.maximum(deg_ref[...], 1.0)
    out_ref[:, :DH] = sum_ref[0, :, :] * r
    out_ref[:, DH:] = sum_ref[1, :, :] * r


def _div_final(sums, deg_col):
    return pl.pallas_call(
        _div_final_body,
        grid=(NV // _OUT_B,),
        in_specs=[
            pl.BlockSpec((NC, _OUT_B, DH), lambda i: (0, i, 0)),
            pl.BlockSpec((_OUT_B, 1), lambda i: (i, 0)),
        ],
        out_specs=pl.BlockSpec((_OUT_B, D), lambda i: (i, 0)),
        out_shape=jax.ShapeDtypeStruct((NV, D), jnp.float32),
    )(sums, deg_col)


def kernel(x, hyperedge_index, W, b):
    v4d = hyperedge_index[0].reshape(NS, IBLK, IROWS, CHUNK)
    e4d = hyperedge_index[1].reshape(NS, IBLK, IROWS, CHUNK)
    z128 = jnp.zeros((ZROWS, DH), jnp.float32)
    ones1d = jnp.ones((CHUNK,), jnp.float32)
    z1d = jnp.zeros((NVP,), jnp.float32)

    xt = _matmul_planes(x, W, b.reshape(1, D))
    e_sum, e_deg, v_deg = _make_sc_stage(True)(xt, v4d, e4d, z128, ones1d, z1d)
    e_feat = _div_mid(e_sum, e_deg.reshape(NVP, 1))
    (v_sum,) = _make_sc_stage(False)(e_feat, e4d, v4d, z128)
    return _div_final(v_sum, v_deg.reshape(NVP, 1))


# speedup vs baseline: 7.4489x; 1.0015x over previous
import functools

import jax
import jax.numpy as jnp
from jax import lax
from jax.experimental import pallas as pl
from jax.experimental.pallas import tpu as pltpu
from jax.experimental.pallas import tpu_sc as plsc

NV = 10000
NNZ = 160000
D = 256
DH = 128
NC = 2
NS = 16
NVP = 10240
DR = NVP // 128
CHUNK = 80
IBLK = 5
IROWS = 25
CPT = NNZ // CHUNK // NS
RPT = NVP // NS
ZROWS = 32


@functools.lru_cache(maxsize=None)
def _make_sc_stage(with_degs: bool):
    mesh = plsc.VectorSubcoreMesh(
        core_axis_name="c", subcore_axis_name="s",
        num_cores=NC, num_subcores=NS)
    out_type = [jax.ShapeDtypeStruct((NC, NVP, DH), jnp.float32)]
    scratch = [
        pltpu.VMEM((IROWS, CHUNK), jnp.int32),
        pltpu.VMEM((IROWS, CHUNK), jnp.int32),
        pltpu.VMEM((CHUNK, DH), jnp.float32),
        pltpu.VMEM((CHUNK, DH), jnp.float32),
        pltpu.VMEM((CHUNK, DH), jnp.float32),
        pltpu.VMEM((ZROWS, DH), jnp.float32),
        pltpu.VMEM_SHARED((NVP, DH), jnp.float32),
        pltpu.SemaphoreType.DMA,
        pltpu.SemaphoreType.DMA,
        pltpu.SemaphoreType.DMA,
        pltpu.SemaphoreType.DMA,
        pltpu.SemaphoreType.DMA,
        pltpu.SemaphoreType.DMA,
    ]
    if with_degs:
        out_type += [jax.ShapeDtypeStruct((NVP,), jnp.float32),
                     jax.ShapeDtypeStruct((NVP,), jnp.float32)]
        scratch += [
            pltpu.VMEM((CHUNK,), jnp.float32),
            pltpu.VMEM((RPT,), jnp.float32),
            pltpu.VMEM_SHARED((NVP,), jnp.float32),
            pltpu.SemaphoreType.DMA,
        ]

    @functools.partial(pl.kernel, mesh=mesh, out_type=out_type,
                       scratch_types=scratch)
    def stage(*refs):
        if with_degs:
            (table, gidx, sidx, z128, ones1d, z1d,
             out, edeg_out, vdeg_out,
             gbuf, sbuf, rows_a, rows_b, rows_c, zbuf, acc,
             gs_a, gs_b, gs_c, ss_a, ss_b, ss_c,
             ones_v, dsum, dacc, dsem) = refs
        else:
            (table, gidx, sidx, z128,
             out, gbuf, sbuf, rows_a, rows_b, rows_c, zbuf, acc,
             gs_a, gs_b, gs_c, ss_a, ss_b, ss_c) = refs
        c = lax.axis_index("c")
        s = lax.axis_index("s")
        pltpu.sync_copy(z128, zbuf)
        base = s * RPT
        for i in range(RPT // ZROWS):
            pltpu.sync_copy(zbuf, acc.at[pl.ds(base + i * ZROWS, ZROWS)])
        if with_degs:
            pltpu.sync_copy(ones1d, ones_v)

            @pl.when(s == 0)
            def _():
                pltpu.sync_copy(z1d, dacc)
        plsc.subcore_barrier()

        def deg_fire(j):
            if not with_degs:
                return

            @pl.when(c == 0)
            def _():
                pltpu.async_copy(ones_v, dacc.at[sbuf.at[j]], dsem, add=True)

            @pl.when(c == 1)
            def _():
                pltpu.async_copy(ones_v, dacc.at[gbuf.at[j]], dsem, add=True)

        def gather(j, buf, sem):
            return pltpu.async_copy(table.at[c].at[gbuf.at[j]], buf, sem)

        def scat(j, buf, sem):
            pltpu.async_copy(buf, acc.at[sbuf.at[j]], sem, add=True)

        def drain_scat(buf, sem):
            pltpu.make_async_copy(buf, acc.at[sbuf.at[0]], sem).wait()

        def blk_body(blk, carry):
            pltpu.sync_copy(gidx.at[s].at[blk], gbuf)
            pltpu.sync_copy(sidx.at[s].at[blk], sbuf)

            g0 = gather(0, rows_a, gs_a)
            g1 = gather(1, rows_b, gs_b)
            g2 = gather(2, rows_c, gs_c)
            g0.wait()
            scat(0, rows_a, ss_a)
            g1.wait()
            scat(1, rows_b, ss_b)
            g2.wait()
            scat(2, rows_c, ss_c)
            deg_fire(0)
            deg_fire(1)
            deg_fire(2)

            def trio_body(t, carry2):
                ja = 3 * t + 3
                jb = 3 * t + 4
                jc = 3 * t + 5
                drain_scat(rows_a, ss_a)
                ga = gather(ja, rows_a, gs_a)
                drain_scat(rows_b, ss_b)
                gb = gather(jb, rows_b, gs_b)
                drain_scat(rows_c, ss_c)
                gc = gather(jc, rows_c, gs_c)
                ga.wait()
                scat(ja, rows_a, ss_a)
                gb.wait()
                scat(jb, rows_b, ss_b)
                gc.wait()
                scat(jc, rows_c, ss_c)
                deg_fire(ja)
                deg_fire(jb)
                deg_fire(jc)
                return carry2

            lax.fori_loop(0, (IROWS - 4) // 3, trio_body, 0)

            drain_scat(rows_a, ss_a)
            gather(IROWS - 1, rows_a, gs_a).wait()
            scat(IROWS - 1, rows_a, ss_a)
            deg_fire(IROWS - 1)
            drain_scat(rows_a, ss_a)
            drain_scat(rows_b, ss_b)
            drain_scat(rows_c, ss_c)
            return carry

        lax.fori_loop(0, IBLK, blk_body, 0)
        if with_degs:
            def deg_drain(j, carry):
                pltpu.make_async_copy(ones_v, dacc.at[sbuf.at[0]],
                                      dsem).wait()
                return carry

            lax.fori_loop(0, CPT, deg_drain, 0)
        plsc.subcore_barrier()

        for i in range(RPT // ZROWS):
            pltpu.sync_copy(acc.at[pl.ds(base + i * ZROWS, ZROWS)], zbuf)
            pltpu.sync_copy(zbuf,
                            out.at[c].at[pl.ds(base + i * ZROWS, ZROWS)])
        if with_degs:
            pltpu.sync_copy(dacc.at[pl.ds(base, RPT)], dsum)

            @pl.when(c == 0)
            def _():
                pltpu.sync_copy(dsum, edeg_out.at[pl.ds(base, RPT)])

            @pl.when(c == 1)
            def _():
                pltpu.sync_copy(dsum, vdeg_out.at[pl.ds(base, RPT)])

    return stage


_DIV_B = 512
_OUT_B = 1000


def _mid_body(sum_ref, deg_ref, w_ref, b_ref, out_ref):
    r = 1.0 / jnp.maximum(deg_ref[...], 1.0)
    xs = jnp.concatenate(
        [sum_ref[0, :, :] * r, sum_ref[1, :, :] * r], axis=1)
    ef = jnp.dot(xs, w_ref[...], precision=lax.Precision.HIGHEST,
                 preferred_element_type=jnp.float32) + b_ref[...]
    out_ref[0, :, :] = ef[:, :DH]
    out_ref[1, :, :] = ef[:, DH:]


def _mid_fused(sums, deg_col, W, b2):
    return pl.pallas_call(
        _mid_body,
        grid=(NVP // _DIV_B,),
        in_specs=[
            pl.BlockSpec((NC, _DIV_B, DH), lambda i: (0, i, 0)),
            pl.BlockSpec((_DIV_B, 1), lambda i: (i, 0)),
            pl.BlockSpec((D, D), lambda i: (0, 0)),
            pl.BlockSpec((1, D), lambda i: (0, 0)),
        ],
        out_specs=pl.BlockSpec((NC, _DIV_B, DH), lambda i: (0, i, 0)),
        out_shape=jax.ShapeDtypeStruct((NC, NVP, DH), jnp.float32),
    )(sums, deg_col, W, b2)


def _div_final_body(sum_ref, deg_ref, out_ref):
    r = 1.0 / jnp.maximum(deg_ref[...], 1.0)
    out_ref[:, :DH] = sum_ref[0, :, :] * r
    out_ref[:, DH:] = sum_ref[1, :, :] * r


def _div_final(sums, deg_col):
    return pl.pallas_call(
        _div_final_body,
        grid=(NV // _OUT_B,),
        in_specs=[
            pl.BlockSpec((NC, _OUT_B, DH), lambda i: (0, i, 0)),
            pl.BlockSpec((_OUT_B, 1), lambda i: (i, 0)),
        ],
        out_specs=pl.BlockSpec((_OUT_B, D), lambda i: (i, 0)),
        out_shape=jax.ShapeDtypeStruct((NV, D), jnp.float32),
    )(sums, deg_col)


def kernel(x, hyperedge_index, W, b):
    v4d = hyperedge_index[0].reshape(NS, IBLK, IROWS, CHUNK)
    e4d = hyperedge_index[1].reshape(NS, IBLK, IROWS, CHUNK)
    z128 = jnp.zeros((ZROWS, DH), jnp.float32)
    ones1d = jnp.ones((CHUNK,), jnp.float32)
    z1d = jnp.zeros((NVP,), jnp.float32)

    x_planes = jnp.stack([x[:, :DH], x[:, DH:]])
    e_sum, e_deg, v_deg = _make_sc_stage(True)(
        x_planes, v4d, e4d, z128, ones1d, z1d)
    e_feat = _mid_fused(e_sum, e_deg.reshape(NVP, 1), W, b.reshape(1, D))
    (v_sum,) = _make_sc_stage(False)(e_feat, e4d, v4d, z128)
    return _div_final(v_sum, v_deg.reshape(NVP, 1))
